# baseline (speedup 1.0000x reference)
import numpy as np
import jax
import jax.numpy as jnp
from jax.experimental import pallas as pl
from jax.experimental.pallas import tpu as pltpu

N_PTS = 100000
BLK = 8192
NB = (N_PTS + BLK - 1) // BLK
NPAD = NB * BLK
BIG = float(np.float32(3.0e38))
PAD_COORD = 1.0e6
F32 = jnp.float32


def _dot(a, b, dims):
    return jax.lax.dot_general(a, b, (dims, ((), ())),
                               preferred_element_type=F32)


def _pointnet_body(xyzb_ref, c_ref, ct_ref, c2x_ref, w1_ref, b1_ref, w2_ref, b2_ref,
                   w3_ref, b3_ref, fw1_ref, fb1_ref, fw2_ref, fb2_ref,
                   fw3_ref, fb3_ref, g1_ref, be1_ref, g2_ref, be2_ref,
                   out_ref):
    C = c_ref[...]
    CT = ct_ref[...]
    pn_row = jnp.sum(CT * CT, axis=0, keepdims=True)
    ir = jax.lax.broadcasted_iota(jnp.int32, (512, 512), 0)
    ic = jax.lax.broadcasted_iota(jnp.int32, (512, 512), 1)
    ident = (ir == ic).astype(F32)
    cn = _dot(ident, pn_row, ((1,), (1,)))

    def body(b, carry):
        run_d, run_x, run_y, run_z = carry
        Xb = xyzb_ref[b]
        pnb = jnp.sum(Xb * Xb, axis=0, keepdims=True)
        Db = (cn + pnb) - _dot(c2x_ref[...], Xb, ((1,), (0,)))
        Dext = jnp.concatenate([run_d, Db], axis=1)
        iota = jax.lax.broadcasted_iota(jnp.int32, (512, 5 + BLK), 1)
        nd, nx, ny, nz = [], [], [], []
        for _ in range(5):
            m = jnp.min(Dext, axis=1, keepdims=True)
            pos = jnp.min(jnp.where(Dext == m, iota, 5 + BLK), axis=1,
                          keepdims=True)
            ohb = iota == pos
            ohf = jnp.where(ohb, 1.0, 0.0)
            ohr = ohf[:, :5]
            cb = _dot(ohf[:, 5:], Xb, ((1,), (1,)))
            nd.append(m)
            nx.append(cb[:, 0:1] + jnp.sum(ohr * run_x, axis=1, keepdims=True))
            ny.append(cb[:, 1:2] + jnp.sum(ohr * run_y, axis=1, keepdims=True))
            nz.append(cb[:, 2:3] + jnp.sum(ohr * run_z, axis=1, keepdims=True))
            Dext = jnp.where(ohb, BIG, Dext)
        return (jnp.concatenate(nd, 1), jnp.concatenate(nx, 1),
                jnp.concatenate(ny, 1), jnp.concatenate(nz, 1))

    init = (jnp.full((512, 5), BIG, F32),
            jnp.zeros((512, 5), F32), jnp.zeros((512, 5), F32),
            jnp.zeros((512, 5), F32))
    _, run_x, run_y, run_z = jax.lax.fori_loop(0, NB, body, init)

    W1 = w1_ref[...]
    m1 = None
    for k in range(5):
        Gk = jnp.concatenate([run_x[:, k:k + 1], run_y[:, k:k + 1],
                              run_z[:, k:k + 1]], axis=1)
        yk = _dot(Gk, W1, ((1,), (0,)))
        m1 = yk if m1 is None else jnp.maximum(m1, yk)
    feat1 = jax.nn.relu(m1 + b1_ref[...])
    x2 = jnp.concatenate([C, feat1], axis=1)

    C2 = C[:256]
    cn2 = cn[:256]
    d2 = (cn2 + pn_row) - 2.0 * _dot(C2, CT, ((1,), (0,)))
    y2 = _dot(x2, w2_ref[...], ((1,), (0,)))
    m2 = None
    D = d2
    iota2 = jax.lax.broadcasted_iota(jnp.int32, (256, 512), 1)
    for _ in range(10):
        m = jnp.min(D, axis=1, keepdims=True)
        eq = D == m
        pos = jnp.min(jnp.where(eq, iota2, 512), axis=1, keepdims=True)
        oh = (iota2 == pos).astype(F32)
        sel = _dot(oh, y2, ((1,), (0,)))
        m2 = sel if m2 is None else jnp.maximum(m2, sel)
        D = jnp.where(oh > 0.0, BIG, D)
    feat2 = jax.nn.relu(m2 + b2_ref[...])
    x3 = jnp.concatenate([C2, feat2], axis=1)

    C3 = C[:128]
    cn3 = cn[:128]
    d3 = (cn3 + pn_row[:, :256]) - 2.0 * _dot(C3, CT[:, :256], ((1,), (0,)))
    y3 = _dot(x3, w3_ref[...], ((1,), (0,)))
    m3 = None
    D = d3
    iota3 = jax.lax.broadcasted_iota(jnp.int32, (128, 256), 1)
    for _ in range(15):
        m = jnp.min(D, axis=1, keepdims=True)
        eq = D == m
        pos = jnp.min(jnp.where(eq, iota3, 256), axis=1, keepdims=True)
        oh = (iota3 == pos).astype(F32)
        sel = _dot(oh, y3, ((1,), (0,)))
        m3 = sel if m3 is None else jnp.maximum(m3, sel)
        D = jnp.where(oh > 0.0, BIG, D)
    feat3 = jax.nn.relu(m3 + b3_ref[...])
    x4 = jnp.concatenate([C3, feat3], axis=1)

    g = jnp.max(x4, axis=0, keepdims=True)
    inv = np.float32(1.0 / np.sqrt(1.0 + 1e-5))
    h = _dot(g, fw1_ref[...], ((1,), (0,))) + fb1_ref[...]
    h = jax.nn.relu(h * inv * g1_ref[...] + be1_ref[...])
    h = _dot(h, fw2_ref[...], ((1,), (0,))) + fb2_ref[...]
    h = jax.nn.relu(h * inv * g2_ref[...] + be2_ref[...])
    h = _dot(h, fw3_ref[...], ((1,), (0,))) + fb3_ref[...]
    out_ref[...] = h


def kernel(points, W1, b1, W2, b2, W3, b3, fc1_W, fc1_b, fc2_W, fc2_b,
           fc3_W, fc3_b, bn1_g, bn1_b, bn2_g, bn2_b):
    pad = jnp.full((NPAD - N_PTS, 3), PAD_COORD, F32)
    xyz_pad = jnp.concatenate([points, pad], axis=0)
    xyzb = jnp.transpose(xyz_pad.T.reshape(3, NB, BLK), (1, 0, 2))
    C = points[:512]

    out = pl.pallas_call(
        _pointnet_body,
        out_shape=jax.ShapeDtypeStruct((1, 12), F32),
    )(xyzb, C, C.T, 2.0 * C, W1, b1.reshape(1, -1), W2, b2.reshape(1, -1),
      W3, b3.reshape(1, -1), fc1_W, fc1_b.reshape(1, -1),
      fc2_W, fc2_b.reshape(1, -1), fc3_W, fc3_b.reshape(1, -1),
      bn1_g.reshape(1, -1), bn1_b.reshape(1, -1),
      bn2_g.reshape(1, -1), bn2_b.reshape(1, -1))
    return out.reshape(4, 3)


# speedup vs baseline: 1.1121x; 1.1121x over previous
import numpy as np
import jax
import jax.numpy as jnp
from jax.experimental import pallas as pl
from jax.experimental.pallas import tpu as pltpu

N_PTS = 100000
BLK = 4096
NB = (N_PTS + BLK - 1) // BLK
NPAD = NB * BLK
BIG = float(np.float32(3.0e38))
PAD_COORD = 1.0e6
F32 = jnp.float32


def _dot(a, b, dims):
    return jax.lax.dot_general(a, b, (dims, ((), ())),
                               preferred_element_type=F32)


def _pointnet_body(xyzb_ref, c_ref, ct_ref, c2x_ref, w1_ref, b1_ref, w2_ref, b2_ref,
                   w3_ref, b3_ref, fw1_ref, fb1_ref, fw2_ref, fb2_ref,
                   fw3_ref, fb3_ref, g1_ref, be1_ref, g2_ref, be2_ref,
                   out_ref):
    C = c_ref[...]
    CT = ct_ref[...]
    pn_row = jnp.sum(CT * CT, axis=0, keepdims=True)
    ir = jax.lax.broadcasted_iota(jnp.int32, (512, 512), 0)
    ic = jax.lax.broadcasted_iota(jnp.int32, (512, 512), 1)
    ident = (ir == ic).astype(F32)
    cn = _dot(ident, pn_row, ((1,), (1,)))

    iota = jax.lax.broadcasted_iota(jnp.int32, (512, 5 + BLK), 1)

    def body(b, carry):
        run_d, run_x, run_y, run_z = carry
        Xb = xyzb_ref[b]
        pnb = jnp.sum(Xb * Xb, axis=0, keepdims=True)
        Db = (cn + pnb) - _dot(c2x_ref[...], Xb, ((1,), (0,)))
        Dext = jnp.concatenate([run_d, Db], axis=1)
        nd, nx, ny, nz = [], [], [], []
        for _ in range(5):
            m = jnp.min(Dext, axis=1, keepdims=True)
            pos = jnp.min(jnp.where(Dext == m, iota, 5 + BLK), axis=1,
                          keepdims=True)
            ohb = iota == pos
            cb = _dot(ohb[:, 5:].astype(jnp.bfloat16), Xb, ((1,), (1,)))
            ohr = jnp.where(ohb[:, :5], 1.0, 0.0)
            nd.append(m)
            nx.append(cb[:, 0:1] + jnp.sum(ohr * run_x, axis=1, keepdims=True))
            ny.append(cb[:, 1:2] + jnp.sum(ohr * run_y, axis=1, keepdims=True))
            nz.append(cb[:, 2:3] + jnp.sum(ohr * run_z, axis=1, keepdims=True))
            Dext = jnp.where(ohb, BIG, Dext)
        return (jnp.concatenate(nd, 1), jnp.concatenate(nx, 1),
                jnp.concatenate(ny, 1), jnp.concatenate(nz, 1))

    init = (jnp.full((512, 5), BIG, F32),
            jnp.zeros((512, 5), F32), jnp.zeros((512, 5), F32),
            jnp.zeros((512, 5), F32))
    _, run_x, run_y, run_z = jax.lax.fori_loop(0, NB, body, init)

    W1 = w1_ref[...]
    m1 = None
    for k in range(5):
        Gk = jnp.concatenate([run_x[:, k:k + 1], run_y[:, k:k + 1],
                              run_z[:, k:k + 1]], axis=1)
        yk = _dot(Gk, W1, ((1,), (0,)))
        m1 = yk if m1 is None else jnp.maximum(m1, yk)
    feat1 = jax.nn.relu(m1 + b1_ref[...])
    x2 = jnp.concatenate([C, feat1], axis=1)

    C2 = C[:256]
    cn2 = cn[:256]
    d2 = (cn2 + pn_row) - 2.0 * _dot(C2, CT, ((1,), (0,)))
    y2 = _dot(x2, w2_ref[...], ((1,), (0,)))
    m2 = None
    D = d2
    iota2 = jax.lax.broadcasted_iota(jnp.int32, (256, 512), 1)
    for _ in range(10):
        m = jnp.min(D, axis=1, keepdims=True)
        eq = D == m
        pos = jnp.min(jnp.where(eq, iota2, 512), axis=1, keepdims=True)
        oh = iota2 == pos
        sel = _dot(oh.astype(jnp.bfloat16), y2, ((1,), (0,)))
        m2 = sel if m2 is None else jnp.maximum(m2, sel)
        D = jnp.where(oh, BIG, D)
    feat2 = jax.nn.relu(m2 + b2_ref[...])
    x3 = jnp.concatenate([C2, feat2], axis=1)

    C3 = C[:128]
    cn3 = cn[:128]
    d3 = (cn3 + pn_row[:, :256]) - 2.0 * _dot(C3, CT[:, :256], ((1,), (0,)))
    y3 = _dot(x3, w3_ref[...], ((1,), (0,)))
    m3 = None
    D = d3
    iota3 = jax.lax.broadcasted_iota(jnp.int32, (128, 256), 1)
    for _ in range(15):
        m = jnp.min(D, axis=1, keepdims=True)
        eq = D == m
        pos = jnp.min(jnp.where(eq, iota3, 256), axis=1, keepdims=True)
        oh = iota3 == pos
        sel = _dot(oh.astype(jnp.bfloat16), y3, ((1,), (0,)))
        m3 = sel if m3 is None else jnp.maximum(m3, sel)
        D = jnp.where(oh, BIG, D)
    feat3 = jax.nn.relu(m3 + b3_ref[...])
    x4 = jnp.concatenate([C3, feat3], axis=1)

    g = jnp.max(x4, axis=0, keepdims=True)
    inv = np.float32(1.0 / np.sqrt(1.0 + 1e-5))
    h = _dot(g, fw1_ref[...], ((1,), (0,))) + fb1_ref[...]
    h = jax.nn.relu(h * inv * g1_ref[...] + be1_ref[...])
    h = _dot(h, fw2_ref[...], ((1,), (0,))) + fb2_ref[...]
    h = jax.nn.relu(h * inv * g2_ref[...] + be2_ref[...])
    h = _dot(h, fw3_ref[...], ((1,), (0,))) + fb3_ref[...]
    out_ref[...] = h


def kernel(points, W1, b1, W2, b2, W3, b3, fc1_W, fc1_b, fc2_W, fc2_b,
           fc3_W, fc3_b, bn1_g, bn1_b, bn2_g, bn2_b):
    pad = jnp.full((NPAD - N_PTS, 3), PAD_COORD, F32)
    xyz_pad = jnp.concatenate([points, pad], axis=0)
    xyzb = jnp.transpose(xyz_pad.T.reshape(3, NB, BLK), (1, 0, 2))
    C = points[:512]

    out = pl.pallas_call(
        _pointnet_body,
        out_shape=jax.ShapeDtypeStruct((1, 12), F32),
    )(xyzb, C, C.T, 2.0 * C, W1, b1.reshape(1, -1), W2, b2.reshape(1, -1),
      W3, b3.reshape(1, -1), fc1_W, fc1_b.reshape(1, -1),
      fc2_W, fc2_b.reshape(1, -1), fc3_W, fc3_b.reshape(1, -1),
      bn1_g.reshape(1, -1), bn1_b.reshape(1, -1),
      bn2_g.reshape(1, -1), bn2_b.reshape(1, -1))
    return out.reshape(4, 3)
